# baseline (speedup 1.0000x reference)
import math

import jax
import jax.numpy as jnp
from jax.experimental import pallas as pl
from jax.experimental.pallas import tpu as pltpu

_WINDOW = 24
_T_RATIO = 0.1
_F_RATIO = 0.1


def _pos_embed(T, D):
    pos = jnp.arange(T, dtype=jnp.float32)[:, None]
    div = jnp.exp(jnp.arange(0, D, 2, dtype=jnp.float32) * (-(math.log(10000.0) / D)))
    pe = jnp.zeros((T, D), jnp.float32)
    pe = pe.at[:, 0::2].set(jnp.sin(pos * div))
    pe = pe.at[:, 1::2].set(jnp.cos(pos * div))
    return pe


def _windowed_sum(e, W):
    B, D, T = e.shape
    pad = jnp.pad(e, ((0, 0), (0, 0), (W - 1, W - 1)))
    cs = jnp.cumsum(pad, axis=-1)
    cs = jnp.concatenate([jnp.zeros((B, D, 1), e.dtype), cs], axis=-1)
    out = cs[..., W:] - cs[..., :-W]
    denom = jnp.concatenate(
        [jnp.arange(1, W, dtype=jnp.float32), jnp.full((T,), float(W), jnp.float32)]
    )
    return out / denom


def _idft_matrices(T, F):
    f = jnp.arange(F, dtype=jnp.int32)[None, :]
    t = jnp.arange(T, dtype=jnp.int32)[:, None]
    m = (f * t) % T
    ang = m.astype(jnp.float32) * jnp.float32(2.0 * math.pi / T)
    w = jnp.where((f == 0) | (f == F - 1), 1.0, 2.0).astype(jnp.float32) / T
    c1 = jnp.cos(ang) * w
    c2 = -jnp.sin(ang) * w
    c2 = c2 * jnp.where((f == 0) | (f == F - 1), 0.0, 1.0)
    return c1, c2


def _gelu(x):
    return 0.5 * x * (1.0 + jax.lax.erf(x * jnp.float32(1.0 / math.sqrt(2.0))))


def _topk_desc(vals, k):
    n = vals.shape[-1]
    iot = jax.lax.broadcasted_iota(jnp.int32, vals.shape, 1)
    kio = jax.lax.broadcasted_iota(jnp.int32, (1, k), 1)

    def step(j, carry):
        cur, idxs, mask = carry
        m = jnp.max(cur)
        idx = jnp.min(jnp.where(cur == m, iot, n))
        idxs = jnp.where(kio == j, idx, idxs)
        sel = iot == idx
        mask = jnp.where(sel, 1.0, mask)
        cur = jnp.where(sel, -jnp.inf, cur)
        return cur, idxs, mask

    idxs0 = jnp.zeros((1, k), jnp.int32)
    mask0 = jnp.zeros(vals.shape, jnp.float32)
    _, idxs, mask = jax.lax.fori_loop(0, k, step, (vals, idxs0, mask0))
    return idxs, mask


def _freq_topk_body(daymag_ref, idxf_ref, maskf_ref):
    idxs, mask = _topk_desc(daymag_ref[0], idxf_ref.shape[-1])
    idxf_ref[0] = idxs
    maskf_ref[0] = mask


def _transform_body(
    score_ref, rows_ref, cnt_ref, ex_ref, tokt_ref, Wt1_ref, bt1_ref, Wt2_ref,
    bt2_ref, cxr_ref, cxi_ref, maskf_ref, tokr_ref, toki_ref, C1_ref, C2_ref,
    Wemb_ref, Wf1_ref, bf1_ref, Wf2_ref, bf2_ref,
    tout_ref, fout_ref, idxt_ref,
):
    ex = ex_ref[0]
    tokt = tokt_ref[...]

    idxs, _ = _topk_desc(score_ref[0], idxt_ref.shape[-1])
    idxt_ref[0] = idxs
    iot_col = jax.lax.broadcasted_iota(jnp.int32, (ex.shape[0], 1), 0)
    mt = jnp.any(iot_col == idxs, axis=1, keepdims=True)

    masked_x = jnp.where(mt != 0, tokt, ex)
    h = jax.lax.dot_general(masked_x, Wt1_ref[...], (((1,), (1,)), ((), ())))
    h = _gelu(h + bt1_ref[...])
    p = jax.lax.dot_general(h, Wt2_ref[...], (((1,), (1,)), ((), ())))
    proj_t = jax.nn.sigmoid(p + bt2_ref[...])
    tout_ref[0] = jnp.where(mt != 0, masked_x, proj_t)

    mf = maskf_ref[0]
    re = jnp.where(mf != 0, tokr_ref[...], cxr_ref[0])
    im = jnp.where(mf != 0, toki_ref[...], cxi_ref[0])
    mx = (
        jax.lax.dot_general(C1_ref[...], re, (((1,), (1,)), ((), ())),
                            precision=jax.lax.Precision.HIGHEST)
        + jax.lax.dot_general(C2_ref[...], im, (((1,), (1,)), ((), ())),
                              precision=jax.lax.Precision.HIGHEST)
    )
    mxc = jnp.dot(mx, Wemb_ref[...])
    fout_ref[0] = mxc

    Wf1 = Wf1_ref[...]
    bf1 = bf1_ref[...]
    Wf2 = Wf2_ref[...]
    bf2 = bf2_ref[0, 0]

    def row_fn(i, _):
        t = rows_ref[0, 0, i]
        v = fout_ref[0, pl.ds(t, 1), :]
        hf = _gelu(Wf1 * v + bf1)
        pf = jax.nn.sigmoid(jnp.dot(Wf2, hf) + bf2)
        fout_ref[0, pl.ds(t, 1), :] = pf
        return 0

    jax.lax.fori_loop(0, cnt_ref[0, 0, 0], row_fn, 0)


def kernel(x, W_emb, b_emb, tok_t, tok_f_real, tok_f_imag,
           Wt1, bt1, Wt2, bt2, Wf1, bf1, Wf2, bf2):
    B, T, C = x.shape
    D = W_emb.shape[0]
    W = _WINDOW
    nmt = int(T * _T_RATIO)
    nmf = int(T * _F_RATIO)

    ex = x @ W_emb.T + b_emb + _pos_embed(T, D)
    exT = jnp.transpose(ex, (0, 2, 1))
    ltr = _windowed_sum(exT, W)
    ltr2 = _windowed_sum(exT ** 2, W)
    ltrd = (ltr2 - ltr ** 2)[..., :T]
    ltrm = ltr[..., :T]
    score = ltrd.sum(axis=1) / (ltrm.sum(axis=1) + 1e-6)

    cx = jnp.fft.rfft(exT, axis=-1)
    mag = jnp.sqrt(cx.real ** 2 + cx.imag ** 2)
    day_mag = mag.mean(axis=1)
    Fn = cx.shape[-1]

    idx_f3, mask_f3 = pl.pallas_call(
        _freq_topk_body,
        grid=(B,),
        in_specs=[pl.BlockSpec((1, 1, Fn), lambda b: (b, 0, 0))],
        out_specs=[
            pl.BlockSpec((1, 1, nmf), lambda b: (b, 0, 0)),
            pl.BlockSpec((1, 1, Fn), lambda b: (b, 0, 0)),
        ],
        out_shape=[
            jax.ShapeDtypeStruct((B, 1, nmf), jnp.int32),
            jax.ShapeDtypeStruct((B, 1, Fn), jnp.float32),
        ],
    )(day_mag.reshape(B, 1, Fn))
    idx_f = idx_f3.reshape(B, nmf)
    mask_f = mask_f3.reshape(B, Fn)

    tm = jnp.fft.irfft(mask_f, n=T, axis=-1) != 0

    need = ~tm
    pos = jnp.cumsum(need, axis=1) - 1
    rows = jnp.full((B, T), 0, jnp.int32).at[
        jnp.arange(B)[:, None], jnp.where(need, pos, T)
    ].set(jnp.broadcast_to(jnp.arange(T, dtype=jnp.int32), (B, T)), mode="drop")
    rows = rows.reshape(B, 1, T)
    cnt = need.sum(axis=1).astype(jnp.int32).reshape(B, 1, 1)

    c1, c2 = _idft_matrices(T, Fn)

    full2 = lambda arr: pl.BlockSpec(arr.shape, lambda b: (0,) * arr.ndim)
    batch3 = lambda s1, s2: pl.BlockSpec((1, s1, s2), lambda b: (b, 0, 0))
    smem = lambda s: pl.BlockSpec((1, 1, s), lambda b: (b, 0, 0), memory_space=pltpu.SMEM)

    tok_t_r = tok_t.reshape(1, D)
    tokr = tok_f_real.reshape(D, 1)
    toki = tok_f_imag.reshape(D, 1)
    bt1_r = bt1.reshape(1, D)
    bt2_r = bt2.reshape(1, D)
    bf1_r = bf1.reshape(D, 1)
    bf2_r = bf2.reshape(1, 1)

    temporal_out, freq_out, idx_t3 = pl.pallas_call(
        _transform_body,
        grid=(B,),
        in_specs=[
            batch3(1, T),
            smem(T),
            smem(1),
            batch3(T, D),
            full2(tok_t_r),
            full2(Wt1), full2(bt1_r), full2(Wt2), full2(bt2_r),
            batch3(D, Fn),
            batch3(D, Fn),
            batch3(1, Fn),
            full2(tokr), full2(toki),
            full2(c1), full2(c2),
            full2(W_emb),
            full2(Wf1), full2(bf1_r), full2(Wf2), full2(bf2_r),
        ],
        out_specs=[batch3(T, D), batch3(T, C), batch3(1, nmt)],
        out_shape=[
            jax.ShapeDtypeStruct((B, T, D), jnp.float32),
            jax.ShapeDtypeStruct((B, T, C), jnp.float32),
            jax.ShapeDtypeStruct((B, 1, nmt), jnp.int32),
        ],
    )(
        score.reshape(B, 1, T), rows, cnt, ex, tok_t_r, Wt1, bt1_r, Wt2, bt2_r,
        cx.real, cx.imag, mask_f.reshape(B, 1, Fn), tokr, toki, c1, c2,
        W_emb, Wf1, bf1_r, Wf2, bf2_r,
    )
    return temporal_out, idx_t3.reshape(B, nmt), freq_out, idx_f


# speedup vs baseline: 1.4929x; 1.4929x over previous
import math

import jax
import jax.numpy as jnp
from jax.experimental import pallas as pl
from jax.experimental.pallas import tpu as pltpu

_WINDOW = 24
_T_RATIO = 0.1
_F_RATIO = 0.1


def _pos_embed(T, D):
    pos = jnp.arange(T, dtype=jnp.float32)[:, None]
    div = jnp.exp(jnp.arange(0, D, 2, dtype=jnp.float32) * (-(math.log(10000.0) / D)))
    pe = jnp.zeros((T, D), jnp.float32)
    pe = pe.at[:, 0::2].set(jnp.sin(pos * div))
    pe = pe.at[:, 1::2].set(jnp.cos(pos * div))
    return pe


def _windowed_sum(e, W):
    B, D, T = e.shape
    pad = jnp.pad(e, ((0, 0), (0, 0), (W - 1, W - 1)))
    cs = jnp.cumsum(pad, axis=-1)
    cs = jnp.concatenate([jnp.zeros((B, D, 1), e.dtype), cs], axis=-1)
    out = cs[..., W:] - cs[..., :-W]
    denom = jnp.concatenate(
        [jnp.arange(1, W, dtype=jnp.float32), jnp.full((T,), float(W), jnp.float32)]
    )
    return out / denom


def _idft_matrices(T, F):
    f = jnp.arange(F, dtype=jnp.int32)[None, :]
    t = jnp.arange(T, dtype=jnp.int32)[:, None]
    m = (f * t) % T
    ang = m.astype(jnp.float32) * jnp.float32(2.0 * math.pi / T)
    w = jnp.where((f == 0) | (f == F - 1), 1.0, 2.0).astype(jnp.float32) / T
    c1 = jnp.cos(ang) * w
    c2 = -jnp.sin(ang) * w
    c2 = c2 * jnp.where((f == 0) | (f == F - 1), 0.0, 1.0)
    return c1, c2


def _gelu(x):
    return 0.5 * x * (1.0 + jax.lax.erf(x * jnp.float32(1.0 / math.sqrt(2.0))))


def _topk_desc(vals, k):
    n = vals.shape[-1]
    iot = jax.lax.broadcasted_iota(jnp.int32, vals.shape, 1)
    kio = jax.lax.broadcasted_iota(jnp.int32, (1, k), 1)

    def step(j, carry):
        cur, idxs, mask = carry
        m = jnp.max(cur)
        idx = jnp.min(jnp.where(cur == m, iot, n))
        idxs = jnp.where(kio == j, idx, idxs)
        sel = iot == idx
        mask = jnp.where(sel, 1.0, mask)
        cur = jnp.where(sel, -jnp.inf, cur)
        return cur, idxs, mask

    idxs0 = jnp.zeros((1, k), jnp.int32)
    mask0 = jnp.zeros(vals.shape, jnp.float32)
    _, idxs, mask = jax.lax.fori_loop(0, k, step, (vals, idxs0, mask0))
    return idxs, mask


def _freq_topk_body(daymag_ref, idxf_ref, maskf_ref):
    idxs, mask = _topk_desc(daymag_ref[0], idxf_ref.shape[-1])
    idxf_ref[0] = idxs
    maskf_ref[0] = mask


def _transform_body(
    maskt_ref, rows_ref, cnt_ref, ex_ref, tokt_ref, Wt1_ref, bt1_ref, Wt2_ref,
    bt2_ref, cxr_ref, cxi_ref, maskf_ref, tokr_ref, toki_ref, C1_ref, C2_ref,
    Wemb_ref, Wf1_ref, bf1_ref, Wf2_ref, bf2_ref,
    tout_ref, fout_ref,
):
    ex = ex_ref[0]
    tokt = tokt_ref[...]

    mt = maskt_ref[0] != 0

    masked_x = jnp.where(mt, tokt, ex)
    h = jax.lax.dot_general(masked_x, Wt1_ref[...], (((1,), (1,)), ((), ())))
    h = _gelu(h + bt1_ref[...])
    p = jax.lax.dot_general(h, Wt2_ref[...], (((1,), (1,)), ((), ())))
    proj_t = jax.nn.sigmoid(p + bt2_ref[...])
    tout_ref[0] = jnp.where(mt, masked_x, proj_t)

    mf = maskf_ref[0]
    re = jnp.where(mf != 0, tokr_ref[...], cxr_ref[0])
    im = jnp.where(mf != 0, toki_ref[...], cxi_ref[0])
    mx = (
        jax.lax.dot_general(C1_ref[...], re, (((1,), (1,)), ((), ())),
                            precision=jax.lax.Precision.HIGHEST)
        + jax.lax.dot_general(C2_ref[...], im, (((1,), (1,)), ((), ())),
                              precision=jax.lax.Precision.HIGHEST)
    )
    mxc = jnp.dot(mx, Wemb_ref[...])
    fout_ref[0] = mxc

    Wf1 = Wf1_ref[...]
    bf1 = bf1_ref[...]
    Wf2 = Wf2_ref[...]
    bf2 = bf2_ref[0, 0]

    def row_fn(i, _):
        t = rows_ref[0, 0, i]
        v = fout_ref[0, pl.ds(t, 1), :]
        hf = _gelu(Wf1 * v + bf1)
        pf = jax.nn.sigmoid(jnp.dot(Wf2, hf) + bf2)
        fout_ref[0, pl.ds(t, 1), :] = pf
        return 0

    jax.lax.fori_loop(0, cnt_ref[0, 0, 0], row_fn, 0)


def kernel(x, W_emb, b_emb, tok_t, tok_f_real, tok_f_imag,
           Wt1, bt1, Wt2, bt2, Wf1, bf1, Wf2, bf2):
    B, T, C = x.shape
    D = W_emb.shape[0]
    W = _WINDOW
    nmt = int(T * _T_RATIO)
    nmf = int(T * _F_RATIO)

    ex = x @ W_emb.T + b_emb + _pos_embed(T, D)
    exT = jnp.transpose(ex, (0, 2, 1))
    ltr = _windowed_sum(exT, W)
    ltr2 = _windowed_sum(exT ** 2, W)
    ltrd = (ltr2 - ltr ** 2)[..., :T]
    ltrm = ltr[..., :T]
    score = ltrd.sum(axis=1) / (ltrm.sum(axis=1) + 1e-6)
    _, idx_t = jax.lax.top_k(score, nmt)
    mask_t = jnp.any(
        jnp.arange(T, dtype=jnp.int32)[None, None, :] == idx_t[:, :, None], axis=1
    ).astype(jnp.float32).reshape(B, T, 1)

    cx = jnp.fft.rfft(exT, axis=-1)
    mag = jnp.sqrt(cx.real ** 2 + cx.imag ** 2)
    day_mag = mag.mean(axis=1)
    Fn = cx.shape[-1]

    _, idx_f = jax.lax.top_k(day_mag, nmf)
    mask_f = jnp.any(
        jnp.arange(Fn, dtype=jnp.int32)[None, None, :] == idx_f[:, :, None], axis=1
    ).astype(jnp.float32)

    tm = jnp.fft.irfft(mask_f, n=T, axis=-1) != 0

    need = ~tm
    pos = jnp.cumsum(need, axis=1) - 1
    rows = jnp.full((B, T), 0, jnp.int32).at[
        jnp.arange(B)[:, None], jnp.where(need, pos, T)
    ].set(jnp.broadcast_to(jnp.arange(T, dtype=jnp.int32), (B, T)), mode="drop")
    rows = rows.reshape(B, 1, T)
    cnt = need.sum(axis=1).astype(jnp.int32).reshape(B, 1, 1)

    c1, c2 = _idft_matrices(T, Fn)

    full2 = lambda arr: pl.BlockSpec(arr.shape, lambda b: (0,) * arr.ndim)
    batch3 = lambda s1, s2: pl.BlockSpec((1, s1, s2), lambda b: (b, 0, 0))
    smem = lambda s: pl.BlockSpec((1, 1, s), lambda b: (b, 0, 0), memory_space=pltpu.SMEM)

    tok_t_r = tok_t.reshape(1, D)
    tokr = tok_f_real.reshape(D, 1)
    toki = tok_f_imag.reshape(D, 1)
    bt1_r = bt1.reshape(1, D)
    bt2_r = bt2.reshape(1, D)
    bf1_r = bf1.reshape(D, 1)
    bf2_r = bf2.reshape(1, 1)

    temporal_out, freq_out = pl.pallas_call(
        _transform_body,
        grid=(B,),
        in_specs=[
            batch3(T, 1),
            smem(T),
            smem(1),
            batch3(T, D),
            full2(tok_t_r),
            full2(Wt1), full2(bt1_r), full2(Wt2), full2(bt2_r),
            batch3(D, Fn),
            batch3(D, Fn),
            batch3(1, Fn),
            full2(tokr), full2(toki),
            full2(c1), full2(c2),
            full2(W_emb),
            full2(Wf1), full2(bf1_r), full2(Wf2), full2(bf2_r),
        ],
        out_specs=[batch3(T, D), batch3(T, C)],
        out_shape=[
            jax.ShapeDtypeStruct((B, T, D), jnp.float32),
            jax.ShapeDtypeStruct((B, T, C), jnp.float32),
        ],
    )(
        mask_t, rows, cnt, ex, tok_t_r, Wt1, bt1_r, Wt2, bt2_r,
        cx.real, cx.imag, mask_f.reshape(B, 1, Fn), tokr, toki, c1, c2,
        W_emb, Wf1, bf1_r, Wf2, bf2_r,
    )
    return temporal_out, idx_t, freq_out, idx_f


# speedup vs baseline: 1.5665x; 1.0493x over previous
import math

import jax
import jax.numpy as jnp
from jax.experimental import pallas as pl
from jax.experimental.pallas import tpu as pltpu

_WINDOW = 24
_T_RATIO = 0.1
_F_RATIO = 0.1


def _pos_embed(T, D):
    pos = jnp.arange(T, dtype=jnp.float32)[:, None]
    div = jnp.exp(jnp.arange(0, D, 2, dtype=jnp.float32) * (-(math.log(10000.0) / D)))
    pe = jnp.zeros((T, D), jnp.float32)
    pe = pe.at[:, 0::2].set(jnp.sin(pos * div))
    pe = pe.at[:, 1::2].set(jnp.cos(pos * div))
    return pe


def _windowed_sum(e, W):
    B, D, T = e.shape
    pad = jnp.pad(e, ((0, 0), (0, 0), (W - 1, W - 1)))
    cs = jnp.cumsum(pad, axis=-1)
    cs = jnp.concatenate([jnp.zeros((B, D, 1), e.dtype), cs], axis=-1)
    out = cs[..., W:] - cs[..., :-W]
    denom = jnp.concatenate(
        [jnp.arange(1, W, dtype=jnp.float32), jnp.full((T,), float(W), jnp.float32)]
    )
    return out / denom


def _idft_matrices(T, F):
    f = jnp.arange(F, dtype=jnp.int32)[None, :]
    t = jnp.arange(T, dtype=jnp.int32)[:, None]
    m = (f * t) % T
    ang = m.astype(jnp.float32) * jnp.float32(2.0 * math.pi / T)
    w = jnp.where((f == 0) | (f == F - 1), 1.0, 2.0).astype(jnp.float32) / T
    c1 = jnp.cos(ang) * w
    c2 = -jnp.sin(ang) * w
    c2 = c2 * jnp.where((f == 0) | (f == F - 1), 0.0, 1.0)
    return c1, c2


def _gelu(x):
    return 0.5 * x * (1.0 + jax.lax.erf(x * jnp.float32(1.0 / math.sqrt(2.0))))


def _topk_desc(vals, k):
    n = vals.shape[-1]
    iot = jax.lax.broadcasted_iota(jnp.int32, vals.shape, 1)
    kio = jax.lax.broadcasted_iota(jnp.int32, (1, k), 1)

    def step(j, carry):
        cur, idxs, mask = carry
        m = jnp.max(cur)
        idx = jnp.min(jnp.where(cur == m, iot, n))
        idxs = jnp.where(kio == j, idx, idxs)
        sel = iot == idx
        mask = jnp.where(sel, 1.0, mask)
        cur = jnp.where(sel, -jnp.inf, cur)
        return cur, idxs, mask

    idxs0 = jnp.zeros((1, k), jnp.int32)
    mask0 = jnp.zeros(vals.shape, jnp.float32)
    _, idxs, mask = jax.lax.fori_loop(0, k, step, (vals, idxs0, mask0))
    return idxs, mask


def _freq_topk_body(daymag_ref, idxf_ref, maskf_ref):
    idxs, mask = _topk_desc(daymag_ref[0], idxf_ref.shape[-1])
    idxf_ref[0] = idxs
    maskf_ref[0] = mask


def _transform_body(
    maskt_ref, rows_ref, cnt_ref, ex_ref, tokt_ref, Wt1_ref, bt1_ref, Wt2_ref,
    bt2_ref, cxr_ref, cxi_ref, maskf_ref, tokr_ref, toki_ref, C1_ref, C2_ref,
    Wemb_ref, Wf1_ref, bf1_ref, Wf2_ref, bf2_ref,
    tout_ref, fout_ref,
):
    ex = ex_ref[0]
    tokt = tokt_ref[...]

    mt = maskt_ref[0] != 0

    masked_x = jnp.where(mt, tokt, ex)
    h = jax.lax.dot_general(masked_x, Wt1_ref[...], (((1,), (1,)), ((), ())))
    h = _gelu(h + bt1_ref[...])
    p = jax.lax.dot_general(h, Wt2_ref[...], (((1,), (1,)), ((), ())))
    proj_t = jax.nn.sigmoid(p + bt2_ref[...])
    tout_ref[0] = jnp.where(mt, masked_x, proj_t)

    mf = maskf_ref[0]
    re = jnp.where(mf != 0, tokr_ref[...], cxr_ref[0])
    im = jnp.where(mf != 0, toki_ref[...], cxi_ref[0])
    mx = (
        jax.lax.dot_general(C1_ref[...], re, (((1,), (1,)), ((), ())),
                            precision=jax.lax.Precision.HIGHEST)
        + jax.lax.dot_general(C2_ref[...], im, (((1,), (1,)), ((), ())),
                              precision=jax.lax.Precision.HIGHEST)
    )
    mxc = jnp.dot(mx, Wemb_ref[...])
    fout_ref[0] = mxc

    Wf1 = Wf1_ref[...]
    bf1 = bf1_ref[...]
    Wf2 = Wf2_ref[...]
    bf2 = bf2_ref[0, 0]

    def row_fn(t, _):
        @pl.when(rows_ref[0, 0, t] != 0)
        def _():
            v = fout_ref[0, pl.ds(t, 1), :]
            hf = _gelu(Wf1 * v + bf1)
            pf = jax.nn.sigmoid(jnp.dot(Wf2, hf) + bf2)
            fout_ref[0, pl.ds(t, 1), :] = pf
        return 0

    @pl.when(cnt_ref[0, 0, 0] > 0)
    def _():
        jax.lax.fori_loop(0, fout_ref.shape[1], row_fn, 0)


def kernel(x, W_emb, b_emb, tok_t, tok_f_real, tok_f_imag,
           Wt1, bt1, Wt2, bt2, Wf1, bf1, Wf2, bf2):
    B, T, C = x.shape
    D = W_emb.shape[0]
    W = _WINDOW
    nmt = int(T * _T_RATIO)
    nmf = int(T * _F_RATIO)

    ex = x @ W_emb.T + b_emb + _pos_embed(T, D)
    exT = jnp.transpose(ex, (0, 2, 1))
    ltr = _windowed_sum(exT, W)
    ltr2 = _windowed_sum(exT ** 2, W)
    ltrd = (ltr2 - ltr ** 2)[..., :T]
    ltrm = ltr[..., :T]
    score = ltrd.sum(axis=1) / (ltrm.sum(axis=1) + 1e-6)
    _, idx_t = jax.lax.top_k(score, nmt)
    mask_t = jnp.any(
        jnp.arange(T, dtype=jnp.int32)[None, None, :] == idx_t[:, :, None], axis=1
    ).astype(jnp.float32).reshape(B, T, 1)

    cx = jnp.fft.rfft(exT, axis=-1)
    mag = jnp.sqrt(cx.real ** 2 + cx.imag ** 2)
    day_mag = mag.mean(axis=1)
    Fn = cx.shape[-1]

    _, idx_f = jax.lax.top_k(day_mag, nmf)
    mask_f = jnp.any(
        jnp.arange(Fn, dtype=jnp.int32)[None, None, :] == idx_f[:, :, None], axis=1
    ).astype(jnp.float32)

    tm = jnp.fft.irfft(mask_f, n=T, axis=-1) != 0

    need = (~tm).astype(jnp.int32)
    rows = need.reshape(B, 1, T)
    cnt = need.sum(axis=1).astype(jnp.int32).reshape(B, 1, 1)

    c1, c2 = _idft_matrices(T, Fn)

    full2 = lambda arr: pl.BlockSpec(arr.shape, lambda b: (0,) * arr.ndim)
    batch3 = lambda s1, s2: pl.BlockSpec((1, s1, s2), lambda b: (b, 0, 0))
    smem = lambda s: pl.BlockSpec((1, 1, s), lambda b: (b, 0, 0), memory_space=pltpu.SMEM)

    tok_t_r = tok_t.reshape(1, D)
    tokr = tok_f_real.reshape(D, 1)
    toki = tok_f_imag.reshape(D, 1)
    bt1_r = bt1.reshape(1, D)
    bt2_r = bt2.reshape(1, D)
    bf1_r = bf1.reshape(D, 1)
    bf2_r = bf2.reshape(1, 1)

    temporal_out, freq_out = pl.pallas_call(
        _transform_body,
        grid=(B,),
        in_specs=[
            batch3(T, 1),
            smem(T),
            smem(1),
            batch3(T, D),
            full2(tok_t_r),
            full2(Wt1), full2(bt1_r), full2(Wt2), full2(bt2_r),
            batch3(D, Fn),
            batch3(D, Fn),
            batch3(1, Fn),
            full2(tokr), full2(toki),
            full2(c1), full2(c2),
            full2(W_emb),
            full2(Wf1), full2(bf1_r), full2(Wf2), full2(bf2_r),
        ],
        out_specs=[batch3(T, D), batch3(T, C)],
        out_shape=[
            jax.ShapeDtypeStruct((B, T, D), jnp.float32),
            jax.ShapeDtypeStruct((B, T, C), jnp.float32),
        ],
    )(
        mask_t, rows, cnt, ex, tok_t_r, Wt1, bt1_r, Wt2, bt2_r,
        cx.real, cx.imag, mask_f.reshape(B, 1, Fn), tokr, toki, c1, c2,
        W_emb, Wf1, bf1_r, Wf2, bf2_r,
    )
    return temporal_out, idx_t, freq_out, idx_f
